# baseline (speedup 1.0000x reference)
import functools

import jax
import jax.numpy as jnp
from jax import lax
from jax.experimental import pallas as pl
from jax.experimental.pallas import tpu as pltpu
from jax.experimental.pallas import tpu_sc as plsc

F_DIM = 64
COL_BLK = 8192
UBLK = 6144
RBLK = 5120
PLEN = 3 * UBLK


def _tc_body(nu, nb, ut_ref, rt_ref, ufl_ref, rfl_ref, wt_ref, b_ref,
             p_ref, iu_ref, ir_ref):
    i = pl.program_id(0)
    wt = wt_ref[...]
    wd = wt[0, :] - wt[1, :]
    db = b_ref[0] - b_ref[1]

    @pl.when(i < nu)
    def _user():
        lhs = wd[:F_DIM].reshape(1, F_DIM)
        p_ref[...] = (jnp.dot(lhs, ut_ref[...]) + db).reshape(UBLK)

    @pl.when(i == nu)
    def _url():
        lhs = wd[F_DIM:].reshape(1, F_DIM)
        pr = jnp.dot(lhs, rt_ref[...]).reshape(RBLK)
        p_ref[...] = jnp.concatenate([pr, jnp.zeros((UBLK - RBLK,), jnp.float32)])

    @pl.when(i < nb)
    def _extract():
        iu_ref[...] = ufl_ref[0, :]
        ir_ref[...] = rfl_ref[0, :] + 2 * UBLK


def _tc_stage(ut_t, rt_t, ufl_t, rfl_t, wt, b2, batch):
    v1, v2 = ut_t.shape[1], rt_t.shape[1]
    nfu, nfr = ufl_t.shape[0], rfl_t.shape[0]
    nu = (v1 + UBLK - 1) // UBLK
    assert v2 <= RBLK and nu * UBLK + UBLK == PLEN
    nb = batch // COL_BLK
    grid = nu + 1
    return pl.pallas_call(
        functools.partial(_tc_body, nu, nb),
        grid=(grid,),
        in_specs=[
            pl.BlockSpec((F_DIM, UBLK), lambda i: (0, jnp.minimum(i, nu - 1))),
            pl.BlockSpec((F_DIM, RBLK), lambda i: (0, 0)),
            pl.BlockSpec((nfu, COL_BLK), lambda i: (0, jnp.minimum(i, nb - 1))),
            pl.BlockSpec((nfr, COL_BLK), lambda i: (0, jnp.minimum(i, nb - 1))),
            pl.BlockSpec((2, 2 * F_DIM), lambda i: (0, 0)),
            pl.BlockSpec(memory_space=pltpu.SMEM),
        ],
        out_specs=[
            pl.BlockSpec((UBLK,), lambda i: (i,)),
            pl.BlockSpec((COL_BLK,), lambda i: (jnp.minimum(i, nb - 1),)),
            pl.BlockSpec((COL_BLK,), lambda i: (jnp.minimum(i, nb - 1),)),
        ],
        out_shape=[
            jax.ShapeDtypeStruct((PLEN,), jnp.float32),
            jax.ShapeDtypeStruct((batch,), jnp.int32),
            jax.ShapeDtypeStruct((batch,), jnp.int32),
        ],
    )(ut_t, rt_t, ufl_t, rfl_t, wt, b2)


def _make_sc_gather(plen, batch):
    nw = 32
    bpw = batch // nw
    mesh = plsc.VectorSubcoreMesh(core_axis_name="c", subcore_axis_name="s")

    @functools.partial(
        pl.kernel,
        mesh=mesh,
        compiler_params=pltpu.CompilerParams(needs_layout_passes=False),
        out_type=jax.ShapeDtypeStruct((2 * batch,), jnp.float32),
        scratch_types=[
            pltpu.VMEM((plen,), jnp.float32),
            pltpu.VMEM((bpw,), jnp.int32),
            pltpu.VMEM((bpw,), jnp.int32),
            pltpu.VMEM((2 * bpw,), jnp.float32),
            pltpu.VMEM_SHARED((plen,), jnp.float32),
            pltpu.SemaphoreType.DMA,
        ],
    )
    def sc_gather(p_hbm, iu_hbm, ir_hbm, out_hbm,
                  p_v, iu_v, ir_v, out_v, p_sh, sem):
        sid = lax.axis_index("s")
        wid = sid * 2 + lax.axis_index("c")
        base = wid * bpw
        nsl = plen // 256
        cps = [
            pltpu.make_async_copy(iu_hbm.at[pl.ds(base, bpw)], iu_v, sem),
            pltpu.make_async_copy(ir_hbm.at[pl.ds(base, bpw)], ir_v, sem),
        ]
        for j in range((nsl + 15) // 16):
            if (j + 1) * 16 <= nsl:
                off = (j * 16 + sid) * 256
                cps.append(pltpu.make_async_copy(
                    p_hbm.at[pl.ds(off, 256)], p_sh.at[pl.ds(off, 256)], sem))
        for cp in cps:
            cp.start()
        rem = nsl % 16

        @pl.when(sid < rem)
        def _tail_slice():
            off = (nsl - rem + sid) * 256
            cp = pltpu.make_async_copy(p_hbm.at[pl.ds(off, 256)],
                                       p_sh.at[pl.ds(off, 256)], sem)
            cp.start()
            cp.wait()

        for cp in cps:
            cp.wait()
        plsc.subcore_barrier()
        cp = pltpu.make_async_copy(p_sh, p_v, sem)
        cp.start()
        cp.wait()
        @plsc.parallel_loop(0, bpw // 16, unroll=4)
        def _chunk(i):
            idxu = iu_v[pl.ds(i * 16, 16)]
            idxr = ir_v[pl.ds(i * 16, 16)]
            u = plsc.load_gather(p_v, [idxu])
            r = plsc.load_gather(p_v, [idxr])
            d = u + r
            p0 = 1.0 / (1.0 + jnp.exp(-d))
            off = (i // 8) * 256 + (i % 8) * 16
            out_v[pl.ds(off, 16)] = p0
            out_v[pl.ds(off + 128, 16)] = 1.0 - p0

        pltpu.sync_copy(out_v, out_hbm.at[pl.ds(2 * base, 2 * bpw)])

    return sc_gather


def kernel(user_f_list, url_f_list, user_table, url_table, W, b):
    batch = user_f_list.shape[0]
    p, iu, ir = _tc_stage(user_table.T, url_table.T,
                          user_f_list.T, url_f_list.T, W.T, b, batch)
    sc = _make_sc_gather(p.shape[0], batch)
    flat = sc(p, iu, ir)
    return flat.reshape(batch // 128, 2, 128).transpose(0, 2, 1).reshape(batch, 2)


# speedup vs baseline: 1.0213x; 1.0213x over previous
import functools

import jax
import jax.numpy as jnp
from jax import lax
from jax.experimental import pallas as pl
from jax.experimental.pallas import tpu as pltpu
from jax.experimental.pallas import tpu_sc as plsc

F_DIM = 64
COL_BLK = 8192
UBLK = 6144
RBLK = 5120


def _tc_body(nu, nb, ut_ref, rt_ref, ufl_ref, rfl_ref, wt_ref, b_ref,
             pu_ref, pr_ref, iu_ref, ir_ref):
    i = pl.program_id(0)
    wt = wt_ref[...]
    wd = wt[0, :] - wt[1, :]
    db = b_ref[0] - b_ref[1]

    @pl.when(i < nu)
    def _user():
        lhs = wd[:F_DIM].reshape(1, F_DIM)
        pu_ref[...] = (jnp.dot(lhs, ut_ref[...]) + db).reshape(UBLK)

    @pl.when(i == 0)
    def _url():
        lhs = wd[F_DIM:].reshape(1, F_DIM)
        pr_ref[...] = jnp.dot(lhs, rt_ref[...]).reshape(RBLK)

    @pl.when(i < nb)
    def _extract():
        iu_ref[...] = ufl_ref[0, :]
        ir_ref[...] = rfl_ref[0, :]


def _tc_stage(ut_t, rt_t, ufl_t, rfl_t, wt, b2, batch):
    v1, v2 = ut_t.shape[1], rt_t.shape[1]
    nfu, nfr = ufl_t.shape[0], rfl_t.shape[0]
    nu = (v1 + UBLK - 1) // UBLK
    assert v2 <= RBLK
    nb = batch // COL_BLK
    grid = max(nu, nb)
    return pl.pallas_call(
        functools.partial(_tc_body, nu, nb),
        grid=(grid,),
        in_specs=[
            pl.BlockSpec((F_DIM, UBLK), lambda i: (0, jnp.minimum(i, nu - 1))),
            pl.BlockSpec((F_DIM, RBLK), lambda i: (0, 0)),
            pl.BlockSpec((nfu, COL_BLK), lambda i: (0, jnp.minimum(i, nb - 1))),
            pl.BlockSpec((nfr, COL_BLK), lambda i: (0, jnp.minimum(i, nb - 1))),
            pl.BlockSpec((2, 2 * F_DIM), lambda i: (0, 0)),
            pl.BlockSpec(memory_space=pltpu.SMEM),
        ],
        out_specs=[
            pl.BlockSpec((UBLK,), lambda i: (jnp.minimum(i, nu - 1),)),
            pl.BlockSpec((RBLK,), lambda i: (0,)),
            pl.BlockSpec((COL_BLK,), lambda i: (jnp.minimum(i, nb - 1),)),
            pl.BlockSpec((COL_BLK,), lambda i: (jnp.minimum(i, nb - 1),)),
        ],
        out_shape=[
            jax.ShapeDtypeStruct((nu * UBLK, ), jnp.float32),
            jax.ShapeDtypeStruct((RBLK,), jnp.float32),
            jax.ShapeDtypeStruct((batch,), jnp.int32),
            jax.ShapeDtypeStruct((batch,), jnp.int32),
        ],
    )(ut_t, rt_t, ufl_t, rfl_t, wt, b2)


def _make_sc_gather(v1p, v2p, batch):
    nw = 32
    bpw = batch // nw
    mesh = plsc.VectorSubcoreMesh(core_axis_name="c", subcore_axis_name="s")

    @functools.partial(
        pl.kernel,
        mesh=mesh,
        compiler_params=pltpu.CompilerParams(needs_layout_passes=False),
        out_type=jax.ShapeDtypeStruct((2 * batch,), jnp.float32),
        scratch_types=[
            pltpu.VMEM((v1p,), jnp.float32),
            pltpu.VMEM((v2p,), jnp.float32),
            pltpu.VMEM((bpw,), jnp.int32),
            pltpu.VMEM((bpw,), jnp.int32),
            pltpu.VMEM((2 * bpw,), jnp.float32),
            pltpu.VMEM_SHARED((v1p,), jnp.float32),
            pltpu.VMEM_SHARED((v2p,), jnp.float32),
            pltpu.SemaphoreType.DMA,
        ],
    )
    def sc_gather(pu_hbm, pr_hbm, iu_hbm, ir_hbm, out_hbm,
                  pu_v, pr_v, iu_v, ir_v, out_v, pu_sh, pr_sh, sem):
        sid = lax.axis_index("s")
        wid = sid * 2 + lax.axis_index("c")
        base = wid * bpw
        c1 = v1p // 16
        ns2 = v2p // 256
        cps = [
            pltpu.make_async_copy(iu_hbm.at[pl.ds(base, bpw)], iu_v, sem),
            pltpu.make_async_copy(ir_hbm.at[pl.ds(base, bpw)], ir_v, sem),
            pltpu.make_async_copy(pu_hbm.at[pl.ds(sid * c1, c1)],
                                  pu_sh.at[pl.ds(sid * c1, c1)], sem),
            pltpu.make_async_copy(pr_hbm.at[pl.ds(sid * 256, 256)],
                                  pr_sh.at[pl.ds(sid * 256, 256)], sem),
        ]
        for cp in cps:
            cp.start()

        @pl.when(sid < ns2 - 16)
        def _extra_pr_slice():
            off = (16 + sid) * 256
            cp = pltpu.make_async_copy(pr_hbm.at[pl.ds(off, 256)],
                                       pr_sh.at[pl.ds(off, 256)], sem)
            cp.start()
            cp.wait()

        for cp in cps:
            cp.wait()
        plsc.subcore_barrier()
        cps = [
            pltpu.make_async_copy(pu_sh, pu_v, sem),
            pltpu.make_async_copy(pr_sh, pr_v, sem),
        ]
        for cp in cps:
            cp.start()
        for cp in cps:
            cp.wait()
        @plsc.parallel_loop(0, bpw // 16, unroll=4)
        def _chunk(i):
            idxu = iu_v[pl.ds(i * 16, 16)]
            idxr = ir_v[pl.ds(i * 16, 16)]
            u = plsc.load_gather(pu_v, [idxu])
            r = plsc.load_gather(pr_v, [idxr])
            d = u + r
            p0 = 1.0 / (1.0 + jnp.exp(-d))
            off = (i // 8) * 256 + (i % 8) * 16
            out_v[pl.ds(off, 16)] = p0
            out_v[pl.ds(off + 128, 16)] = 1.0 - p0

        pltpu.sync_copy(out_v, out_hbm.at[pl.ds(2 * base, 2 * bpw)])

    return sc_gather


def kernel(user_f_list, url_f_list, user_table, url_table, W, b):
    batch = user_f_list.shape[0]
    pu, pr, iu, ir = _tc_stage(user_table.T, url_table.T,
                               user_f_list.T, url_f_list.T, W.T, b, batch)
    sc = _make_sc_gather(pu.shape[0], pr.shape[0], batch)
    flat = sc(pu, pr, iu, ir)
    return flat.reshape(batch // 128, 2, 128).transpose(0, 2, 1).reshape(batch, 2)


# speedup vs baseline: 1.0265x; 1.0050x over previous
import functools

import jax
import jax.numpy as jnp
from jax import lax
from jax.experimental import pallas as pl
from jax.experimental.pallas import tpu as pltpu
from jax.experimental.pallas import tpu_sc as plsc

F_DIM = 64
COL_BLK = 8192
UBLK = 6144
RBLK = 5120


def _tc_body(nu, nb, ut_ref, rt_ref, ufl_ref, rfl_ref, wt_ref, b_ref,
             pu_ref, pr_ref, iu_ref, ir_ref):
    i = pl.program_id(0)
    wt = wt_ref[...]
    wd = wt[0, :] - wt[1, :]
    db = b_ref[0] - b_ref[1]

    @pl.when(i < nu)
    def _user():
        lhs = wd[:F_DIM].reshape(1, F_DIM)
        pu_ref[...] = (jnp.dot(lhs, ut_ref[...]) + db).reshape(UBLK)

    @pl.when(i == 0)
    def _url():
        lhs = wd[F_DIM:].reshape(1, F_DIM)
        pr_ref[...] = jnp.dot(lhs, rt_ref[...]).reshape(RBLK)

    @pl.when(i < nb)
    def _extract():
        iu_ref[...] = ufl_ref[0, :]
        ir_ref[...] = rfl_ref[0, :]


def _tc_stage(ut_t, rt_t, ufl_t, rfl_t, wt, b2, batch):
    v1, v2 = ut_t.shape[1], rt_t.shape[1]
    nfu, nfr = ufl_t.shape[0], rfl_t.shape[0]
    nu = (v1 + UBLK - 1) // UBLK
    assert v2 <= RBLK
    nb = batch // COL_BLK
    grid = max(nu, nb)
    return pl.pallas_call(
        functools.partial(_tc_body, nu, nb),
        grid=(grid,),
        in_specs=[
            pl.BlockSpec((F_DIM, UBLK), lambda i: (0, jnp.minimum(i, nu - 1))),
            pl.BlockSpec((F_DIM, RBLK), lambda i: (0, 0)),
            pl.BlockSpec((nfu, COL_BLK), lambda i: (0, jnp.minimum(i, nb - 1))),
            pl.BlockSpec((nfr, COL_BLK), lambda i: (0, jnp.minimum(i, nb - 1))),
            pl.BlockSpec((2, 2 * F_DIM), lambda i: (0, 0)),
            pl.BlockSpec(memory_space=pltpu.SMEM),
        ],
        out_specs=[
            pl.BlockSpec((UBLK,), lambda i: (jnp.minimum(i, nu - 1),)),
            pl.BlockSpec((RBLK,), lambda i: (0,)),
            pl.BlockSpec((COL_BLK,), lambda i: (jnp.minimum(i, nb - 1),)),
            pl.BlockSpec((COL_BLK,), lambda i: (jnp.minimum(i, nb - 1),)),
        ],
        out_shape=[
            jax.ShapeDtypeStruct((nu * UBLK, ), jnp.float32),
            jax.ShapeDtypeStruct((RBLK,), jnp.float32),
            jax.ShapeDtypeStruct((batch,), jnp.int32),
            jax.ShapeDtypeStruct((batch,), jnp.int32),
        ],
    )(ut_t, rt_t, ufl_t, rfl_t, wt, b2)


def _make_sc_gather(v1p, v2p, batch):
    nw = 32
    bpw = batch // nw
    mesh = plsc.VectorSubcoreMesh(core_axis_name="c", subcore_axis_name="s")

    @functools.partial(
        pl.kernel,
        mesh=mesh,
        compiler_params=pltpu.CompilerParams(needs_layout_passes=False),
        out_type=jax.ShapeDtypeStruct((2 * batch,), jnp.float32),
        scratch_types=[
            pltpu.VMEM((v1p,), jnp.float32),
            pltpu.VMEM((v2p,), jnp.float32),
            pltpu.VMEM((bpw,), jnp.int32),
            pltpu.VMEM((bpw,), jnp.int32),
            pltpu.VMEM((2 * bpw,), jnp.float32),
            pltpu.VMEM_SHARED((v1p,), jnp.float32),
            pltpu.VMEM_SHARED((v2p,), jnp.float32),
            pltpu.SemaphoreType.DMA,
        ],
    )
    def sc_gather(pu_hbm, pr_hbm, iu_hbm, ir_hbm, out_hbm,
                  pu_v, pr_v, iu_v, ir_v, out_v, pu_sh, pr_sh, sem):
        sid = lax.axis_index("s")
        wid = sid * 2 + lax.axis_index("c")
        base = wid * bpw
        c1 = v1p // 16
        ns2 = v2p // 256
        cps = [
            pltpu.make_async_copy(iu_hbm.at[pl.ds(base, bpw)], iu_v, sem),
            pltpu.make_async_copy(ir_hbm.at[pl.ds(base, bpw)], ir_v, sem),
            pltpu.make_async_copy(pu_hbm.at[pl.ds(sid * c1, c1)],
                                  pu_sh.at[pl.ds(sid * c1, c1)], sem),
            pltpu.make_async_copy(pr_hbm.at[pl.ds(sid * 256, 256)],
                                  pr_sh.at[pl.ds(sid * 256, 256)], sem),
        ]
        for cp in cps:
            cp.start()

        @pl.when(sid < ns2 - 16)
        def _extra_pr_slice():
            off = (16 + sid) * 256
            cp = pltpu.make_async_copy(pr_hbm.at[pl.ds(off, 256)],
                                       pr_sh.at[pl.ds(off, 256)], sem)
            cp.start()
            cp.wait()

        for cp in cps:
            cp.wait()
        plsc.subcore_barrier()
        cps = [
            pltpu.make_async_copy(pu_sh, pu_v, sem),
            pltpu.make_async_copy(pr_sh, pr_v, sem),
        ]
        for cp in cps:
            cp.start()
        for cp in cps:
            cp.wait()
        @plsc.parallel_loop(0, bpw // 16, unroll=8)
        def _chunk(i):
            idxu = iu_v[pl.ds(i * 16, 16)]
            idxr = ir_v[pl.ds(i * 16, 16)]
            u = plsc.load_gather(pu_v, [idxu])
            r = plsc.load_gather(pr_v, [idxr])
            d = u + r
            p0 = 1.0 / (1.0 + jnp.exp(-d))
            off = (i // 8) * 256 + (i % 8) * 16
            out_v[pl.ds(off, 16)] = p0
            out_v[pl.ds(off + 128, 16)] = 1.0 - p0

        pltpu.sync_copy(out_v, out_hbm.at[pl.ds(2 * base, 2 * bpw)])

    return sc_gather


def kernel(user_f_list, url_f_list, user_table, url_table, W, b):
    batch = user_f_list.shape[0]
    pu, pr, iu, ir = _tc_stage(user_table.T, url_table.T,
                               user_f_list.T, url_f_list.T, W.T, b, batch)
    sc = _make_sc_gather(pu.shape[0], pr.shape[0], batch)
    flat = sc(pu, pr, iu, ir)
    return flat.reshape(batch // 128, 2, 128).transpose(0, 2, 1).reshape(batch, 2)
